# Initial kernel scaffold; baseline (speedup 1.0000x reference)
import jax
import jax.numpy as jnp
from jax.experimental import pallas as pl


def kernel(x, W_feat, W_dir, bn_w, bn_b, W_pool):
    raise NotImplementedError("write your pallas kernel here")


# speedup vs baseline: 2.9349x; 2.9349x over previous
import jax
import jax.numpy as jnp
from jax.experimental import pallas as pl

EPS = 1e-6
K = 20
TN = 256
TB = 128
F = K * TB
NEG = -float("inf")


def _knn_kernel(xt_ref, xp_ref, nbr_ref):
    xt = xt_ref[0]
    xp = xp_ref[0]
    xsq = jnp.sum(xp * xp, axis=0, keepdims=True)
    dist = 2.0 * jax.lax.dot(xt, xp, preferred_element_type=jnp.float32) - xsq
    n = dist.shape[1]
    jio = jax.lax.broadcasted_iota(jnp.int32, dist.shape, 1)
    for k in range(K):
        m = jnp.max(dist, axis=1, keepdims=True)
        cand = jnp.where(dist == m, jio, n)
        jmin = jnp.min(cand, axis=1, keepdims=True)
        onehot = jio == jmin
        oh = onehot.astype(jnp.float32)
        nbr_k = jax.lax.dot_general(
            xp, oh, (((1,), (1,)), ((), ())),
            preferred_element_type=jnp.float32,
            precision=jax.lax.Precision.HIGHEST)
        nbr_ref[0, k] = nbr_k
        dist = jnp.where(onehot, NEG, dist)


def _r16(v):
    return v.astype(jnp.bfloat16).astype(jnp.float32)


def _stats_kernel(e_ref, c_ref, wf_ref, s_ref):
    wf = _r16(wf_ref[...])
    w0 = wf[:, 0:1]
    w1 = wf[:, 1:2]
    nsq = jnp.zeros((wf.shape[0], F), jnp.float32)
    for d in range(3):
        e = _r16(e_ref[0, 0, d:d + 1, :])
        c = _r16(c_ref[0, 0, d:d + 1, :])
        p = w0 * e + w1 * c
        nsq = nsq + p * p
    nrm = jnp.sqrt(nsq) + EPS
    s1 = jnp.sum(nrm, axis=1, keepdims=True)
    s2 = jnp.sum(nrm * nrm, axis=1, keepdims=True)
    s_ref[0, 0] = jnp.concatenate([s1, s2], axis=1)


def _conv_kernel(e_ref, c_ref, wf_ref, wd_ref, wp_ref, al_ref, be_ref, o_ref):
    wf = _r16(wf_ref[...])
    wd = _r16(wd_ref[...])
    wp = wp_ref[...]
    al = al_ref[...]
    be = be_ref[...]
    e = [_r16(e_ref[0, 0, d:d + 1, :]) for d in range(3)]
    c = [_r16(c_ref[0, 0, d:d + 1, :]) for d in range(3)]
    p = [wf[:, 0:1] * e[d] + wf[:, 1:2] * c[d] for d in range(3)]
    nsq = p[0] * p[0] + p[1] * p[1] + p[2] * p[2]
    nrm = jnp.sqrt(nsq) + EPS
    fac = al + be / nrm
    ph = [p[d] * fac for d in range(3)]
    dd = [wd[:, 0:1] * e[d] + wd[:, 1:2] * c[d] for d in range(3)]
    dot = ph[0] * dd[0] + ph[1] * dd[1] + ph[2] * dd[2]
    dsq = dd[0] * dd[0] + dd[1] * dd[1] + dd[2] * dd[2]
    g = 0.8 * jnp.where(dot < 0.0, dot / (dsq + EPS), 0.0)
    h = [ph[d] - g * dd[d] for d in range(3)]
    d2 = [jax.lax.dot(wp, h[d], preferred_element_type=jnp.float32)
          for d in range(3)]
    t = h[0] * d2[0] + h[1] * d2[1] + h[2] * d2[2]
    m = t[:, 0:TB]
    for k in range(1, K):
        m = jnp.maximum(m, t[:, k * TB:(k + 1) * TB])
    found = jnp.zeros_like(m, dtype=jnp.bool_)
    out = [jnp.zeros_like(m) for _ in range(3)]
    for k in range(K):
        sl = slice(k * TB, (k + 1) * TB)
        ismax = jnp.logical_and(t[:, sl] == m, jnp.logical_not(found))
        found = jnp.logical_or(found, ismax)
        for d in range(3):
            out[d] = jnp.where(ismax, h[d][:, sl], out[d])
    for d in range(3):
        o_ref[0, 0, d] = out[d]


def kernel(x, W_feat, W_dir, bn_w, bn_b, W_pool):
    B, _, N = x.shape
    NT = N // TB
    CO = W_feat.shape[0]

    xpad = jnp.pad(x, ((0, 0), (0, 5), (0, 0)))
    xt = jnp.transpose(xpad, (0, 2, 1))

    nbr = pl.pallas_call(
        _knn_kernel,
        grid=(B, N // TN),
        in_specs=[
            pl.BlockSpec((1, TN, 8), lambda b, t: (b, t, 0)),
            pl.BlockSpec((1, 8, N), lambda b, t: (b, 0, 0)),
        ],
        out_specs=pl.BlockSpec((1, K, 8, TN), lambda b, t: (b, 0, 0, t)),
        out_shape=jax.ShapeDtypeStruct((B, K, 8, N), jnp.float32),
    )(xt, xpad)

    nbr3 = nbr[:, :, :3, :]
    e4 = nbr3 - x[:, None, :, :]
    c4 = jnp.broadcast_to(x[:, None, :, :], e4.shape)

    def to_tiled(a):
        a = a.reshape(B, K, 3, NT, TB)
        a = jnp.transpose(a, (0, 3, 2, 1, 4))
        return a.reshape(B, NT, 3, F)

    E = to_tiled(e4)
    C = to_tiled(c4)

    ec_spec = pl.BlockSpec((1, 1, 3, F), lambda b, t: (b, t, 0, 0))
    w2_spec = pl.BlockSpec((CO, 2), lambda b, t: (0, 0))

    S = pl.pallas_call(
        _stats_kernel,
        grid=(B, NT),
        in_specs=[ec_spec, ec_spec, w2_spec],
        out_specs=pl.BlockSpec((1, 1, CO, 2), lambda b, t: (b, t, 0, 0)),
        out_shape=jax.ShapeDtypeStruct((B, NT, CO, 2), jnp.float32),
    )(E, C, W_feat)

    tot = jnp.sum(S, axis=(0, 1))
    cnt = jnp.float32(B * N * K)
    mean = tot[:, 0] / cnt
    var = tot[:, 1] / cnt - mean * mean
    inv = bn_w / jnp.sqrt(var + 1e-5)
    alpha = inv[:, None]
    beta2 = (bn_b - mean * inv)[:, None]

    O = pl.pallas_call(
        _conv_kernel,
        grid=(B, NT),
        in_specs=[
            ec_spec, ec_spec, w2_spec, w2_spec,
            pl.BlockSpec((CO, CO), lambda b, t: (0, 0)),
            pl.BlockSpec((CO, 1), lambda b, t: (0, 0)),
            pl.BlockSpec((CO, 1), lambda b, t: (0, 0)),
        ],
        out_specs=pl.BlockSpec((1, 1, 3, CO, TB), lambda b, t: (b, t, 0, 0, 0)),
        out_shape=jax.ShapeDtypeStruct((B, NT, 3, CO, TB), jnp.float32),
    )(E, C, W_feat, W_dir, W_pool, alpha, beta2)

    out = jnp.transpose(O, (0, 3, 2, 1, 4)).reshape(B, CO, 3, N)
    return out


# speedup vs baseline: 6.2244x; 2.1208x over previous
import jax
import jax.numpy as jnp
from jax.experimental import pallas as pl

EPS = 1e-6
K = 20
TN = 256
SUB = 32
TB = 128
F = K * TB
NEG = -float("inf")


def _knn_kernel(xt_ref, xp_ref, x3_ref, nbr_ref):
    xt = xt_ref[0]
    xp = xp_ref[0]
    x3 = x3_ref[0]
    xsq = jnp.sum(xp * xp, axis=0, keepdims=True)
    dist = 2.0 * jax.lax.dot(xt, xp, preferred_element_type=jnp.float32) - xsq
    n = dist.shape[1]
    jio = jax.lax.broadcasted_iota(jnp.int32, dist.shape, 1)
    for k in range(K):
        m = jnp.max(dist, axis=1, keepdims=True)
        cand = jnp.where(dist == m, jio, n)
        jmin = jnp.min(cand, axis=1, keepdims=True)
        onehot = jio == jmin
        oh = jnp.where(onehot, 1.0, 0.0).astype(jnp.bfloat16)
        r = jax.lax.dot_general(
            x3, oh, (((1,), (1,)), ((), ())),
            preferred_element_type=jnp.float32)
        nbr_ref[0, k] = (r[0:8] + r[8:16]) + r[16:24]
        dist = jnp.where(onehot, NEG, dist)


def _r16(v):
    return v.astype(jnp.bfloat16).astype(jnp.float32)


def _stats_kernel(e_ref, c_ref, wf_ref, s_ref):
    wf = _r16(wf_ref[...])
    w0 = wf[:, 0:1]
    w1 = wf[:, 1:2]
    nsq = jnp.zeros((wf.shape[0], F), jnp.float32)
    for d in range(3):
        e = _r16(e_ref[0, 0, d:d + 1, :])
        c = _r16(c_ref[0, 0, d:d + 1, :])
        p = w0 * e + w1 * c
        nsq = nsq + p * p
    nrm = jnp.sqrt(nsq) + EPS
    s1 = jnp.sum(nrm, axis=1, keepdims=True)
    s2 = jnp.sum(nrm * nrm, axis=1, keepdims=True)
    s_ref[0, 0] = jnp.concatenate([s1, s2], axis=1)


def _conv_kernel(e_ref, c_ref, wf_ref, wd_ref, wp_ref, al_ref, be_ref, o_ref):
    wf = _r16(wf_ref[...])
    wd = _r16(wd_ref[...])
    wp = wp_ref[...]
    al = al_ref[...]
    be = be_ref[...]
    e = [_r16(e_ref[0, 0, d:d + 1, :]) for d in range(3)]
    c = [_r16(c_ref[0, 0, d:d + 1, :]) for d in range(3)]
    p = [wf[:, 0:1] * e[d] + wf[:, 1:2] * c[d] for d in range(3)]
    nsq = p[0] * p[0] + p[1] * p[1] + p[2] * p[2]
    nrm = jnp.sqrt(nsq) + EPS
    fac = al + be / nrm
    ph = [p[d] * fac for d in range(3)]
    dd = [wd[:, 0:1] * e[d] + wd[:, 1:2] * c[d] for d in range(3)]
    dot = ph[0] * dd[0] + ph[1] * dd[1] + ph[2] * dd[2]
    dsq = dd[0] * dd[0] + dd[1] * dd[1] + dd[2] * dd[2]
    g = 0.8 * jnp.where(dot < 0.0, dot / (dsq + EPS), 0.0)
    h = [ph[d] - g * dd[d] for d in range(3)]
    d2 = [jax.lax.dot(wp, h[d], preferred_element_type=jnp.float32)
          for d in range(3)]
    t = h[0] * d2[0] + h[1] * d2[1] + h[2] * d2[2]
    m = t[:, 0:TB]
    for k in range(1, K):
        m = jnp.maximum(m, t[:, k * TB:(k + 1) * TB])
    found = jnp.zeros_like(m, dtype=jnp.bool_)
    out = [jnp.zeros_like(m) for _ in range(3)]
    for k in range(K):
        sl = slice(k * TB, (k + 1) * TB)
        ismax = jnp.logical_and(t[:, sl] == m, jnp.logical_not(found))
        found = jnp.logical_or(found, ismax)
        for d in range(3):
            out[d] = jnp.where(ismax, h[d][:, sl], out[d])
    for d in range(3):
        o_ref[0, 0, d] = out[d]


def kernel(x, W_feat, W_dir, bn_w, bn_b, W_pool):
    B, _, N = x.shape
    NT = N // TB
    CO = W_feat.shape[0]

    xpad = jnp.pad(x, ((0, 0), (0, 5), (0, 0)))
    xt = jnp.transpose(xpad, (0, 2, 1))
    xhi = xpad.astype(jnp.bfloat16)
    r1 = xpad - xhi.astype(jnp.float32)
    xmid = r1.astype(jnp.bfloat16)
    xlo = (r1 - xmid.astype(jnp.float32)).astype(jnp.bfloat16)
    x3 = jnp.concatenate([xhi, xmid, xlo], axis=1)

    nbr = pl.pallas_call(
        _knn_kernel,
        grid=(B, N // TN),
        in_specs=[
            pl.BlockSpec((1, TN, 8), lambda b, t: (b, t, 0)),
            pl.BlockSpec((1, 8, N), lambda b, t: (b, 0, 0)),
            pl.BlockSpec((1, 24, N), lambda b, t: (b, 0, 0)),
        ],
        out_specs=pl.BlockSpec((1, K, 8, TN), lambda b, t: (b, 0, 0, t)),
        out_shape=jax.ShapeDtypeStruct((B, K, 8, N), jnp.float32),
    )(xt, xpad, x3)

    nbr3 = nbr[:, :, :3, :]
    e4 = nbr3 - x[:, None, :, :]
    c4 = jnp.broadcast_to(x[:, None, :, :], e4.shape)

    def to_tiled(a):
        a = a.reshape(B, K, 3, NT, TB)
        a = jnp.transpose(a, (0, 3, 2, 1, 4))
        return a.reshape(B, NT, 3, F)

    E = to_tiled(e4)
    C = to_tiled(c4)

    ec_spec = pl.BlockSpec((1, 1, 3, F), lambda b, t: (b, t, 0, 0))
    w2_spec = pl.BlockSpec((CO, 2), lambda b, t: (0, 0))

    S = pl.pallas_call(
        _stats_kernel,
        grid=(B, NT),
        in_specs=[ec_spec, ec_spec, w2_spec],
        out_specs=pl.BlockSpec((1, 1, CO, 2), lambda b, t: (b, t, 0, 0)),
        out_shape=jax.ShapeDtypeStruct((B, NT, CO, 2), jnp.float32),
    )(E, C, W_feat)

    tot = jnp.sum(S, axis=(0, 1))
    cnt = jnp.float32(B * N * K)
    mean = tot[:, 0] / cnt
    var = tot[:, 1] / cnt - mean * mean
    inv = bn_w / jnp.sqrt(var + 1e-5)
    alpha = inv[:, None]
    beta2 = (bn_b - mean * inv)[:, None]

    O = pl.pallas_call(
        _conv_kernel,
        grid=(B, NT),
        in_specs=[
            ec_spec, ec_spec, w2_spec, w2_spec,
            pl.BlockSpec((CO, CO), lambda b, t: (0, 0)),
            pl.BlockSpec((CO, 1), lambda b, t: (0, 0)),
            pl.BlockSpec((CO, 1), lambda b, t: (0, 0)),
        ],
        out_specs=pl.BlockSpec((1, 1, 3, CO, TB), lambda b, t: (b, t, 0, 0, 0)),
        out_shape=jax.ShapeDtypeStruct((B, NT, 3, CO, TB), jnp.float32),
    )(E, C, W_feat, W_dir, W_pool, alpha, beta2)

    out = jnp.transpose(O, (0, 3, 2, 1, 4)).reshape(B, CO, 3, N)
    return out
